# baseline (speedup 1.0000x reference)
import functools

import jax
import jax.numpy as jnp
from jax.experimental import pallas as pl

F32 = jnp.float32


def _dot(a, b):
    return jax.lax.dot_general(a, b, (((1,), (0,)), ((), ())),
                               preferred_element_type=F32)


def _pick_chunk(p, target=2048):
    best = 1
    for t in range(1, p + 1):
        if p % t == 0 and p // t <= target:
            return p // t
    return best


def _pick_rows(h, w, target=2048):
    best = 1
    for r in range(h, 0, -1):
        if h % r == 0 and r * w <= target:
            return r
    return best


def _up2(x):
    n, h, w, c = x.shape
    x = jnp.broadcast_to(x[:, :, None, :, None, :], (n, h, 2, w, 2, c))
    return x.reshape(n, 2 * h, 2 * w, c)


def _c1x1_body(relu, has_add, x_ref, w_ref, b_ref, *rest):
    if has_add:
        a_ref, o_ref = rest
    else:
        (o_ref,) = rest
    acc = _dot(x_ref[0], w_ref[:]) + b_ref[:]
    if relu:
        acc = jnp.maximum(acc, 0.0)
    if has_add:
        acc = acc + a_ref[0]
    o_ref[0] = acc


def _conv1x1(x, w, b, add=None, relu=False):
    n, p, c = x.shape
    o = w.shape[1]
    ch = _pick_chunk(p)
    t = p // ch
    in_specs = [
        pl.BlockSpec((1, ch, c), lambda i, j: (i, j, 0)),
        pl.BlockSpec((c, o), lambda i, j: (0, 0)),
        pl.BlockSpec((1, o), lambda i, j: (0, 0)),
    ]
    args = [x, w, b.reshape(1, o)]
    if add is not None:
        in_specs.append(pl.BlockSpec((1, ch, o), lambda i, j: (i, j, 0)))
        args.append(add)
    return pl.pallas_call(
        functools.partial(_c1x1_body, relu, add is not None),
        grid=(n, t),
        in_specs=in_specs,
        out_specs=pl.BlockSpec((1, ch, o), lambda i, j: (i, j, 0)),
        out_shape=jax.ShapeDtypeStruct((n, p, o), F32),
    )(*args)


def _s1_body(r, w, relu, has_add, x_ref, w_ref, b_ref, *rest):
    if has_add:
        a_ref, o_ref = rest
    else:
        (o_ref,) = rest
    row0 = pl.program_id(1) * r
    acc = None
    for ky in range(3):
        for kx in range(3):
            xs = x_ref[0, pl.ds(row0 + ky, r), pl.ds(kx, w), :]
            xs = xs.reshape(r * w, xs.shape[-1])
            d = _dot(xs, w_ref[ky * 3 + kx])
            acc = d if acc is None else acc + d
    acc = acc + b_ref[:]
    if relu:
        acc = jnp.maximum(acc, 0.0)
    if has_add:
        acc = acc + a_ref[0]
    o_ref[0] = acc


def _conv3x3_s1(x, w, b, add=None, relu=False):
    n, h, wd, c = x.shape
    o = w.shape[0]
    xp = jnp.pad(x, ((0, 0), (1, 1), (1, 1), (0, 0)))
    w9 = jnp.transpose(w, (2, 3, 1, 0)).reshape(9, c, o)
    r = _pick_rows(h, wd)
    t = h // r
    in_specs = [
        pl.BlockSpec((1, h + 2, wd + 2, c), lambda i, j: (i, 0, 0, 0)),
        pl.BlockSpec((9, c, o), lambda i, j: (0, 0, 0)),
        pl.BlockSpec((1, o), lambda i, j: (0, 0)),
    ]
    args = [xp, w9, b.reshape(1, o)]
    if add is not None:
        in_specs.append(pl.BlockSpec((1, r * wd, o), lambda i, j: (i, j, 0)))
        args.append(add)
    return pl.pallas_call(
        functools.partial(_s1_body, r, wd, relu, add is not None),
        grid=(n, t),
        in_specs=in_specs,
        out_specs=pl.BlockSpec((1, r * wd, o), lambda i, j: (i, j, 0)),
        out_shape=jax.ShapeDtypeStruct((n, h * wd, o), F32),
    )(*args)


def _s2_body(r, wo, c, relu, x_ref, w_ref, b_ref, o_ref):
    row0 = pl.program_id(1) * r
    acc = None
    for ky in range(3):
        a, oy = ky % 2, ky // 2
        for kx in range(3):
            b, ox = kx % 2, kx // 2
            xs = x_ref[0, pl.ds(row0 + oy, r), a, pl.ds(ox, wo), pl.ds(b * c, c)]
            xs = xs.reshape(r * wo, c)
            d = _dot(xs, w_ref[ky * 3 + kx])
            acc = d if acc is None else acc + d
    acc = acc + b_ref[:]
    if relu:
        acc = jnp.maximum(acc, 0.0)
    o_ref[0] = acc


def _conv3x3_s2(x, w, b, relu=True):
    n, h, wd, c = x.shape
    o = w.shape[0]
    ho, wo = h // 2, wd // 2
    xp = jnp.pad(x, ((0, 0), (0, 2), (0, 2), (0, 0)))
    xv = xp.reshape(n, ho + 1, 2, wo + 1, 2 * c)
    w9 = jnp.transpose(w, (2, 3, 1, 0)).reshape(9, c, o)
    r = _pick_rows(ho, wo)
    t = ho // r
    return pl.pallas_call(
        functools.partial(_s2_body, r, wo, c, relu),
        grid=(n, t),
        in_specs=[
            pl.BlockSpec((1, ho + 1, 2, wo + 1, 2 * c),
                         lambda i, j: (i, 0, 0, 0, 0)),
            pl.BlockSpec((9, c, o), lambda i, j: (0, 0, 0)),
            pl.BlockSpec((1, o), lambda i, j: (0, 0)),
        ],
        out_specs=pl.BlockSpec((1, r * wo, o), lambda i, j: (i, j, 0)),
        out_shape=jax.ShapeDtypeStruct((n, ho * wo, o), F32),
    )(xv, w9, b.reshape(1, o))


def _first_conv(x, w, b):
    n, h, wd, c = x.shape
    o = w.shape[0]
    ho, wo = h // 2, wd // 2
    xp = jnp.pad(x, ((0, 0), (0, 2), (0, 2), (0, 0)))
    pl_ = {(0, 0): xp[:, 0::2, 0::2, :], (0, 1): xp[:, 0::2, 1::2, :],
           (1, 0): xp[:, 1::2, 0::2, :], (1, 1): xp[:, 1::2, 1::2, :]}
    taps = []
    for ky in range(3):
        a, oy = ky % 2, ky // 2
        for kx in range(3):
            bb, ox = kx % 2, kx // 2
            taps.append(pl_[(a, bb)][:, oy:oy + ho, ox:ox + wo, :])
    x27 = jnp.concatenate(taps, axis=-1).reshape(n, ho * wo, 9 * c)
    w27 = jnp.transpose(w, (2, 3, 1, 0)).reshape(9 * c, o)
    return _conv1x1(x27, w27, b, relu=True)


def kernel(x, params):
    p = params
    n = x.shape[0]
    xt = jnp.transpose(x, (0, 2, 3, 1))

    c1 = _first_conv(xt, p['enc_w1'], p['enc_b1'])
    c1 = c1.reshape(n, 192, 192, 64)
    c2 = _conv3x3_s2(c1, p['enc_w2'], p['enc_b2'])
    c2r = c2.reshape(n, 96, 96, 64)
    c3 = _conv3x3_s2(c2r, p['enc_w3'], p['enc_b3'])
    c3r = c3.reshape(n, 48, 48, 128)
    c4 = _conv3x3_s2(c3r, p['enc_w4'], p['enc_b4'])
    c4r = c4.reshape(n, 24, 24, 256)
    c5 = _conv3x3_s2(c4r, p['enc_w5'], p['enc_b5'])

    lat = lambda lvl: (p['lat_w%d' % lvl][:, :, 0, 0].T, p['lat_b%d' % lvl])
    w5, b5 = lat(5)
    p5 = _conv1x1(c5, w5, b5)
    p5r = p5.reshape(n, 12, 12, 256)
    w4, b4 = lat(4)
    p4 = _conv1x1(c4, w4, b4, add=_up2(p5r).reshape(n, 576, 256))
    p4r = p4.reshape(n, 24, 24, 256)
    w3, b3 = lat(3)
    p3 = _conv1x1(c3, w3, b3, add=_up2(p4r).reshape(n, 2304, 256))
    p3r = p3.reshape(n, 48, 48, 256)
    w2, b2 = lat(2)
    p2 = _conv1x1(c2, w2, b2, add=_up2(p3r).reshape(n, 9216, 256))
    p2r = p2.reshape(n, 96, 96, 256)

    m5 = _conv3x3_s1(p5r, p['seg_w5'], p['seg_b5'], relu=True)
    m5r = m5.reshape(n, 12, 12, 128)
    m4 = _conv3x3_s1(p4r, p['seg_w4'], p['seg_b4'], relu=True,
                     add=_up2(m5r).reshape(n, 576, 128))
    m4r = m4.reshape(n, 24, 24, 128)
    m3 = _conv3x3_s1(p3r, p['seg_w3'], p['seg_b3'], relu=True,
                     add=_up2(m4r).reshape(n, 1152 * 2, 128))
    m3r = m3.reshape(n, 48, 48, 128)
    m2 = _conv3x3_s1(p2r, p['seg_w2'], p['seg_b2'], relu=True,
                     add=_up2(m3r).reshape(n, 9216, 128))

    wf = p['final_w'][:, :, 0, 0].T
    z = _conv1x1(m2, wf, p['final_b'])
    z = z.reshape(n, 96, 96, 4)
    z = jnp.transpose(z, (0, 3, 1, 2))
    z = jnp.broadcast_to(z[:, :, :, None, :, None], (n, 4, 96, 4, 96, 4))
    return z.reshape(n, 4, 384, 384)


# speedup vs baseline: 1.0110x; 1.0110x over previous
import functools

import jax
import jax.numpy as jnp
from jax.experimental import pallas as pl

F32 = jnp.float32
BF16 = jnp.bfloat16


def _dot(a, b):
    return jax.lax.dot_general(a.astype(BF16), b, (((1,), (0,)), ((), ())),
                               preferred_element_type=F32)


def _pick_chunk(p, target=2048):
    best = 1
    for t in range(1, p + 1):
        if p % t == 0 and p // t <= target:
            return p // t
    return best


def _pick_rows(h, w, target=2048):
    best = 1
    for r in range(h, 0, -1):
        if h % r == 0 and r * w <= target:
            return r
    return best


def _up2(x):
    n, h, w, c = x.shape
    x = jnp.broadcast_to(x[:, :, None, :, None, :], (n, h, 2, w, 2, c))
    return x.reshape(n, 2 * h, 2 * w, c)


def _c1x1_body(relu, has_add, x_ref, w_ref, b_ref, *rest):
    if has_add:
        a_ref, o_ref = rest
    else:
        (o_ref,) = rest
    acc = _dot(x_ref[0], w_ref[:]) + b_ref[:]
    if relu:
        acc = jnp.maximum(acc, 0.0)
    if has_add:
        acc = acc + a_ref[0]
    o_ref[0] = acc


def _conv1x1(x, w, b, add=None, relu=False):
    n, p, c = x.shape
    o = w.shape[1]
    ch = _pick_chunk(p)
    t = p // ch
    in_specs = [
        pl.BlockSpec((1, ch, c), lambda i, j: (i, j, 0)),
        pl.BlockSpec((c, o), lambda i, j: (0, 0)),
        pl.BlockSpec((1, o), lambda i, j: (0, 0)),
    ]
    args = [x, w.astype(BF16), b.reshape(1, o)]
    if add is not None:
        in_specs.append(pl.BlockSpec((1, ch, o), lambda i, j: (i, j, 0)))
        args.append(add)
    return pl.pallas_call(
        functools.partial(_c1x1_body, relu, add is not None),
        grid=(n, t),
        in_specs=in_specs,
        out_specs=pl.BlockSpec((1, ch, o), lambda i, j: (i, j, 0)),
        out_shape=jax.ShapeDtypeStruct((n, p, o), F32),
    )(*args)


def _s1_body(r, w, relu, has_add, x_ref, w_ref, b_ref, *rest):
    if has_add:
        a_ref, o_ref = rest
    else:
        (o_ref,) = rest
    row0 = pl.program_id(1) * r
    acc = None
    for ky in range(3):
        for kx in range(3):
            xs = x_ref[0, pl.ds(row0 + ky, r), pl.ds(kx, w), :]
            xs = xs.reshape(r * w, xs.shape[-1])
            d = _dot(xs, w_ref[ky * 3 + kx])
            acc = d if acc is None else acc + d
    acc = acc + b_ref[:]
    if relu:
        acc = jnp.maximum(acc, 0.0)
    if has_add:
        acc = acc + a_ref[0]
    o_ref[0] = acc


def _conv3x3_s1(x, w, b, add=None, relu=False):
    n, h, wd, c = x.shape
    o = w.shape[0]
    xp = jnp.pad(x, ((0, 0), (1, 1), (1, 7), (0, 0)))
    w9 = jnp.transpose(w, (2, 3, 1, 0)).reshape(9, c, o).astype(BF16)
    r = _pick_rows(h, wd)
    t = h // r
    in_specs = [
        pl.BlockSpec((1, h + 2, wd + 8, c), lambda i, j: (i, 0, 0, 0)),
        pl.BlockSpec((9, c, o), lambda i, j: (0, 0, 0)),
        pl.BlockSpec((1, o), lambda i, j: (0, 0)),
    ]
    args = [xp, w9, b.reshape(1, o)]
    if add is not None:
        in_specs.append(pl.BlockSpec((1, r * wd, o), lambda i, j: (i, j, 0)))
        args.append(add)
    return pl.pallas_call(
        functools.partial(_s1_body, r, wd, relu, add is not None),
        grid=(n, t),
        in_specs=in_specs,
        out_specs=pl.BlockSpec((1, r * wd, o), lambda i, j: (i, j, 0)),
        out_shape=jax.ShapeDtypeStruct((n, h * wd, o), F32),
    )(*args)


def _s2_body(r, wo, c, relu, x_ref, w_ref, b_ref, o_ref):
    row0 = pl.program_id(1) * r
    acc = None
    for ky in range(3):
        a, oy = ky % 2, ky // 2
        xa = x_ref[0, pl.ds(row0 + oy, r), a, pl.ds(0, wo), :]
        d = _dot(xa.reshape(r * wo, 2 * c), w_ref[ky, pl.ds(0, 2 * c), :])
        acc = d if acc is None else acc + d
        xb = x_ref[0, pl.ds(row0 + oy, r), a, pl.ds(1, wo), pl.ds(0, c)]
        acc = acc + _dot(xb.reshape(r * wo, c), w_ref[ky, pl.ds(2 * c, c), :])
    acc = acc + b_ref[:]
    if relu:
        acc = jnp.maximum(acc, 0.0)
    o_ref[0] = acc


def _conv3x3_s2(x, w, b, relu=True):
    n, h, wd, c = x.shape
    o = w.shape[0]
    ho, wo = h // 2, wd // 2
    xp = jnp.pad(x, ((0, 0), (0, 2), (0, 16), (0, 0)))
    xv = xp.reshape(n, ho + 1, 2, wo + 8, 2 * c)
    wt = jnp.transpose(w, (2, 3, 1, 0))
    w9 = jnp.concatenate([wt[:, 0], wt[:, 1], wt[:, 2]], axis=1).astype(BF16)
    r = _pick_rows(ho, wo)
    t = ho // r
    return pl.pallas_call(
        functools.partial(_s2_body, r, wo, c, relu),
        grid=(n, t),
        in_specs=[
            pl.BlockSpec((1, ho + 1, 2, wo + 8, 2 * c),
                         lambda i, j: (i, 0, 0, 0, 0)),
            pl.BlockSpec((3, 3 * c, o), lambda i, j: (0, 0, 0)),
            pl.BlockSpec((1, o), lambda i, j: (0, 0)),
        ],
        out_specs=pl.BlockSpec((1, r * wo, o), lambda i, j: (i, j, 0)),
        out_shape=jax.ShapeDtypeStruct((n, ho * wo, o), F32),
    )(xv, w9, b.reshape(1, o))


def _first_conv(x, w, b):
    n, h, wd, c = x.shape
    o = w.shape[0]
    ho, wo = h // 2, wd // 2
    xp = jnp.pad(x, ((0, 0), (0, 2), (0, 2), (0, 0)))
    pl_ = {(0, 0): xp[:, 0::2, 0::2, :], (0, 1): xp[:, 0::2, 1::2, :],
           (1, 0): xp[:, 1::2, 0::2, :], (1, 1): xp[:, 1::2, 1::2, :]}
    taps = []
    for ky in range(3):
        a, oy = ky % 2, ky // 2
        for kx in range(3):
            bb, ox = kx % 2, kx // 2
            taps.append(pl_[(a, bb)][:, oy:oy + ho, ox:ox + wo, :])
    x27 = jnp.concatenate(taps, axis=-1).reshape(n, ho * wo, 9 * c)
    w27 = jnp.transpose(w, (2, 3, 1, 0)).reshape(9 * c, o)
    return _conv1x1(x27, w27, b, relu=True)


def kernel(x, params):
    p = params
    n = x.shape[0]
    xt = jnp.transpose(x, (0, 2, 3, 1))

    c1 = _first_conv(xt, p['enc_w1'], p['enc_b1'])
    c1 = c1.reshape(n, 192, 192, 64)
    c2 = _conv3x3_s2(c1, p['enc_w2'], p['enc_b2'])
    c2r = c2.reshape(n, 96, 96, 64)
    c3 = _conv3x3_s2(c2r, p['enc_w3'], p['enc_b3'])
    c3r = c3.reshape(n, 48, 48, 128)
    c4 = _conv3x3_s2(c3r, p['enc_w4'], p['enc_b4'])
    c4r = c4.reshape(n, 24, 24, 256)
    c5 = _conv3x3_s2(c4r, p['enc_w5'], p['enc_b5'])

    lat = lambda lvl: (p['lat_w%d' % lvl][:, :, 0, 0].T, p['lat_b%d' % lvl])
    w5, b5 = lat(5)
    p5 = _conv1x1(c5, w5, b5)
    p5r = p5.reshape(n, 12, 12, 256)
    w4, b4 = lat(4)
    p4 = _conv1x1(c4, w4, b4, add=_up2(p5r).reshape(n, 576, 256))
    p4r = p4.reshape(n, 24, 24, 256)
    w3, b3 = lat(3)
    p3 = _conv1x1(c3, w3, b3, add=_up2(p4r).reshape(n, 2304, 256))
    p3r = p3.reshape(n, 48, 48, 256)
    w2, b2 = lat(2)
    p2 = _conv1x1(c2, w2, b2, add=_up2(p3r).reshape(n, 9216, 256))
    p2r = p2.reshape(n, 96, 96, 256)

    m5 = _conv3x3_s1(p5r, p['seg_w5'], p['seg_b5'], relu=True)
    m5r = m5.reshape(n, 12, 12, 128)
    m4 = _conv3x3_s1(p4r, p['seg_w4'], p['seg_b4'], relu=True,
                     add=_up2(m5r).reshape(n, 576, 128))
    m4r = m4.reshape(n, 24, 24, 128)
    m3 = _conv3x3_s1(p3r, p['seg_w3'], p['seg_b3'], relu=True,
                     add=_up2(m4r).reshape(n, 1152 * 2, 128))
    m3r = m3.reshape(n, 48, 48, 128)
    m2 = _conv3x3_s1(p2r, p['seg_w2'], p['seg_b2'], relu=True,
                     add=_up2(m3r).reshape(n, 9216, 128))

    wf = p['final_w'][:, :, 0, 0].T
    z = _conv1x1(m2, wf, p['final_b'])
    z = z.reshape(n, 96, 96, 4)
    z = jnp.transpose(z, (0, 3, 1, 2))
    z = jnp.broadcast_to(z[:, :, :, None, :, None], (n, 4, 96, 4, 96, 4))
    return z.reshape(n, 4, 384, 384)
